# baseline (speedup 1.0000x reference)
import functools

import jax
import jax.numpy as jnp
from jax import lax
from jax.experimental import pallas as pl
from jax.experimental.pallas import tpu as pltpu
from jax.experimental.pallas import tpu_sc as plsc

B, S, D, L, V = 64, 512, 768, 300, 100000
N = B * L
NC, NS, LANES = 2, 16, 16
NW = NC * NS
RPW = B // NW
LP = 304
EPW_PAD = RPW * LP
NCHUNK = EPW_PAD // LANES
GRPS = tuple((l0, 40) for l0 in range(0, 280, 40)) + ((280, 24),)
GMAX = 40
NBUF = 3
DCH = D // LANES


def _extract_i32(vec, j):
    io = lax.iota(jnp.int32, LANES)
    return jnp.sum(jnp.where(io == j, vec, 0))


def _extract_f32(vec, j):
    io = lax.iota(jnp.int32, LANES)
    return jnp.sum(jnp.where(io == j, vec, jnp.float32(0)))


def _sc_body(ernie_hbm, meta_hbm, table_hbm, out_hbm,
             mv, tloc, clsl, stl, enl, w0l, buf2, ebuf, obuf8, qrow, acc,
             sem, gsem0, gsem1, gsem2, ssem0, ssem1, ssem2):
    wid = lax.axis_index("s") * NC + lax.axis_index("c")
    io = lax.iota(jnp.int32, LANES)
    zero16 = jnp.zeros((LANES,), jnp.float32)
    gsems = (gsem0, gsem1, gsem2)
    ssems = (ssem0, ssem1, ssem2)

    pltpu.sync_copy(meta_hbm.at[wid], mv)

    def up(ch, carry):
        o = ch * LANES
        tloc[pl.ds(o, LANES)] = mv[0, pl.ds(o, LANES)]
        clsl[pl.ds(o, LANES)] = mv[1, pl.ds(o, LANES)]
        stl[pl.ds(o, LANES)] = mv[2, pl.ds(o, LANES)]
        enl[pl.ds(o, LANES)] = mv[3, pl.ds(o, LANES)]
        w0l[pl.ds(o, LANES)] = mv[4, pl.ds(o, LANES)]
        return carry

    lax.fori_loop(0, NCHUNK, up, 0)

    groups = [(r, l0, gl) for r in range(RPW) for (l0, gl) in GRPS]
    ng = len(groups)

    def gstart(i):
        r, l0, gl = groups[i]
        return pltpu.async_copy(
            table_hbm.at[tloc.at[pl.ds(r * LP + l0, gl)]],
            buf2.at[i % NBUF, pl.ds(0, gl)], gsems[i % NBUF])

    def sstart(i):
        r, l0, gl = groups[i]
        return pltpu.async_copy(
            buf2.at[i % NBUF, pl.ds(0, gl)],
            out_hbm.at[wid * RPW + r, pl.ds(l0, gl)], ssems[i % NBUF])

    gh = {0: gstart(0), 1: gstart(1)}
    sh = {}
    for i in range(ng):
        gh[i].wait()
        if i + 2 < ng:
            if i - 1 >= 0:
                sh[i - 1].wait()
            gh[i + 2] = gstart(i + 2)
        sh[i] = sstart(i)
    for i in range(max(0, ng - 3), ng):
        sh[i].wait()

    def write_row_to_out(b_s, l_s, src):
        g8 = (l_s // 8) * 8
        rr = l_s - g8
        pltpu.sync_copy(out_hbm.at[b_s, pl.ds(g8, 8)], obuf8)
        for r in range(8):
            @pl.when(rr == r)
            def _cp():
                def ck(k, c):
                    o = k * LANES
                    obuf8[r, pl.ds(o, LANES)] = src[pl.ds(o, LANES)]
                    return c
                lax.fori_loop(0, DCH, ck, 0)
        pltpu.sync_copy(obuf8, out_hbm.at[b_s, pl.ds(g8, 8)])

    def handle_lane(cls_s, st_s, en_s, w0_s, b_s, l_s):
        @pl.when(cls_s == 1)
        def _single():
            s8 = (st_s // 8) * 8
            sr = st_s - s8
            pltpu.sync_copy(ernie_hbm.at[b_s, pl.ds(s8, 8)], obuf8)
            for r in range(8):
                @pl.when(sr == r)
                def _cp():
                    def ck(k, c):
                        o = k * LANES
                        qrow[pl.ds(o, LANES)] = obuf8[r, pl.ds(o, LANES)]
                        return c
                    lax.fori_loop(0, DCH, ck, 0)
            write_row_to_out(b_s, l_s, qrow)

        @pl.when(cls_s == 2)
        def _attn():
            pltpu.async_copy(
                table_hbm.at[jnp.full((LANES,), w0_s, jnp.int32)],
                ebuf, sem).wait()

            def qk(k, c):
                o = k * LANES
                qrow[pl.ds(o, LANES)] = ebuf[0, pl.ds(o, LANES)]
                acc[pl.ds(o, LANES)] = zero16
                return c
            lax.fori_loop(0, DCH, qk, 0)

            c0 = st_s // LANES
            c1 = (en_s - 1) // LANES

            def chunk(c, carry):
                m_s, z_s = carry
                pltpu.sync_copy(ernie_hbm.at[b_s, pl.ds(c * LANES, LANES)],
                                ebuf)
                pos = c * LANES + io
                valid = (pos >= st_s) & (pos < en_s)
                sv = jnp.full((LANES,), -1e30, jnp.float32)
                for p in range(LANES):
                    def dk(k, pv):
                        o = k * LANES
                        return pv + (ebuf[p, pl.ds(o, LANES)]
                                     * qrow[pl.ds(o, LANES)])
                    part = lax.fori_loop(0, DCH, dk, zero16)
                    sp = jnp.sum(part)
                    sv = jnp.where(io == p, sp, sv)
                sv = jnp.where(valid, sv, jnp.float32(-1e30))
                mc = jnp.max(sv)
                m_new = jnp.maximum(m_s, mc)
                pe = jnp.exp(sv - m_new)
                pe = jnp.where(valid, pe, jnp.float32(0))
                ssum = jnp.sum(pe)
                scale_v = jnp.exp(jnp.full((LANES,), m_s - m_new))
                z_new = z_s * jnp.max(scale_v) + ssum

                def sk(k, c2):
                    o = k * LANES
                    acc[pl.ds(o, LANES)] = acc[pl.ds(o, LANES)] * scale_v
                    return c2
                lax.fori_loop(0, DCH, sk, 0)
                for p in range(LANES):
                    wp = _extract_f32(pe, p)

                    def ak(k, c3):
                        o = k * LANES
                        acc[pl.ds(o, LANES)] = (
                            acc[pl.ds(o, LANES)]
                            + ebuf[p, pl.ds(o, LANES)] * wp)
                        return c3
                    lax.fori_loop(0, DCH, ak, 0)
                return (m_new, z_new)

            _, z_fin = lax.fori_loop(
                c0, c1 + 1, chunk, (jnp.float32(-1e30), jnp.float32(0)))
            zinv_v = jnp.ones((LANES,), jnp.float32) / jnp.full(
                (LANES,), z_fin)

            def nk(k, c4):
                o = k * LANES
                acc[pl.ds(o, LANES)] = acc[pl.ds(o, LANES)] * zinv_v
                return c4
            lax.fori_loop(0, DCH, nk, 0)
            write_row_to_out(b_s, l_s, acc)

    def chunk_scan(ch, carry):
        cvec = clsl[pl.ds(ch * LANES, LANES)]

        @pl.when(jnp.max(cvec) > 0)
        def _special_chunk():
            svec = stl[pl.ds(ch * LANES, LANES)]
            evec = enl[pl.ds(ch * LANES, LANES)]
            wvec = w0l[pl.ds(ch * LANES, LANES)]

            def lane(p, c):
                cls_s = _extract_i32(cvec, p)

                @pl.when(cls_s > 0)
                def _go():
                    st_s = _extract_i32(svec, p)
                    en_s = _extract_i32(evec, p)
                    w0_s = _extract_i32(wvec, p)
                    lidx = ch * LANES + p
                    b_s = wid * RPW + lidx // LP
                    l_s = lidx % LP
                    handle_lane(cls_s, st_s, en_s, w0_s, b_s, l_s)
                return c
            lax.fori_loop(0, LANES, lane, 0)
        return carry

    lax.fori_loop(0, NCHUNK, chunk_scan, 0)


def _make_call():
    mesh = plsc.VectorSubcoreMesh(
        core_axis_name="c", subcore_axis_name="s",
        num_cores=NC, num_subcores=NS)

    @functools.partial(
        pl.kernel,
        out_type=jax.ShapeDtypeStruct((B, LP, D), jnp.float32),
        mesh=mesh,
        compiler_params=pltpu.CompilerParams(
            use_tc_tiling_on_sc=True, needs_layout_passes=False),
        scratch_types=[
            pltpu.VMEM((8, EPW_PAD), jnp.int32),
            pltpu.VMEM((EPW_PAD,), jnp.int32),
            pltpu.VMEM((EPW_PAD,), jnp.int32),
            pltpu.VMEM((EPW_PAD,), jnp.int32),
            pltpu.VMEM((EPW_PAD,), jnp.int32),
            pltpu.VMEM((EPW_PAD,), jnp.int32),
            pltpu.VMEM((NBUF, GMAX, D), jnp.float32),
            pltpu.VMEM((LANES, D), jnp.float32),
            pltpu.VMEM((8, D), jnp.float32),
            pltpu.VMEM((D,), jnp.float32),
            pltpu.VMEM((D,), jnp.float32),
            pltpu.SemaphoreType.DMA,
            pltpu.SemaphoreType.DMA,
            pltpu.SemaphoreType.DMA,
            pltpu.SemaphoreType.DMA,
            pltpu.SemaphoreType.DMA,
            pltpu.SemaphoreType.DMA,
            pltpu.SemaphoreType.DMA,
        ],
    )
    def call(ernie_hbm, meta_hbm, table_hbm, out_hbm, *scratch):
        _sc_body(ernie_hbm, meta_hbm, table_hbm, out_hbm, *scratch)

    return call


_sc_call = _make_call()


def kernel(ernie_output, word_index, table):
    w0 = word_index[:, :, 0]
    start = word_index[:, :, 1]
    end = word_index[:, :, 2]
    span = end - start

    is_br = (end < S) & (span <= 0)
    has_break = jnp.any(is_br, axis=1)
    jb = jnp.argmax(is_br, axis=1)
    jidx = jnp.arange(L, dtype=jnp.int32)[None, :]
    use_break = has_break[:, None] & (jidx >= jb[:, None])
    w0b = w0[jnp.arange(B), jb]

    notb = ~use_break
    attn = notb & (end < S) & (span > 1)
    single = notb & (end < S) & (span == 1)
    cls = attn.astype(jnp.int32) * 2 + single.astype(jnp.int32)

    tidx = jnp.where(use_break, w0b[:, None], w0)
    tidx = jnp.where(cls > 0, 0, tidx).astype(jnp.int32)
    tidx = jnp.zeros_like(tidx)
    startc = jnp.clip(start, 0, S - 1).astype(jnp.int32)

    def shape_w(a):
        return jnp.pad(a.astype(jnp.int32).reshape(NW, RPW, L),
                       ((0, 0), (0, 0), (0, LP - L))).reshape(NW, EPW_PAD)

    z = jnp.zeros((NW, EPW_PAD), jnp.int32)
    meta = jnp.stack(
        [shape_w(tidx), shape_w(cls), shape_w(startc), shape_w(end),
         shape_w(w0), z, z, z], axis=1)

    return _sc_call(ernie_output, meta, table)[:, :L, :]


# speedup vs baseline: 5.3201x; 5.3201x over previous
import functools

import jax
import jax.numpy as jnp
from jax import lax
from jax.experimental import pallas as pl
from jax.experimental.pallas import tpu as pltpu
from jax.experimental.pallas import tpu_sc as plsc

B, S, D, L, V = 64, 512, 768, 300, 100000
N = B * L
NC, NS, LANES = 2, 16, 16
NW = NC * NS
RPW = B // NW
LP = 320
EPW_PAD = RPW * LP
NCHUNK = EPW_PAD // LANES
GMAX = 32
GRPS = tuple((l0, GMAX) for l0 in range(0, LP, GMAX))
NBUF = 3
BIG = 1 << 30
DCH = D // LANES


def _extract_i32(vec, j):
    io = lax.iota(jnp.int32, LANES)
    return jnp.sum(jnp.where(io == j, vec, 0))


def _extract_f32(vec, j):
    io = lax.iota(jnp.int32, LANES)
    return jnp.sum(jnp.where(io == j, vec, jnp.float32(0)))


def _sc_body(ernie_hbm, meta_hbm, table_hbm, out_hbm,
             mv, tloc, clsl, stl, enl, w0l, buf2, bkbuf, ebuf, obuf8,
             qrow, acc, sem, gsem0, gsem1, gsem2, ssem0, ssem1, ssem2):
    wid = lax.axis_index("s") * NC + lax.axis_index("c")
    io = lax.iota(jnp.int32, LANES)
    zero16 = jnp.zeros((LANES,), jnp.float32)
    gsems = (gsem0, gsem1, gsem2)
    ssems = (ssem0, ssem1, ssem2)

    pltpu.sync_copy(meta_hbm.at[wid], mv)

    def up(ch, carry):
        o = ch * LANES
        tloc[pl.ds(o, LANES)] = mv[0, pl.ds(o, LANES)]
        clsl[pl.ds(o, LANES)] = mv[1, pl.ds(o, LANES)]
        stl[pl.ds(o, LANES)] = mv[2, pl.ds(o, LANES)]
        enl[pl.ds(o, LANES)] = mv[3, pl.ds(o, LANES)]
        w0l[pl.ds(o, LANES)] = mv[4, pl.ds(o, LANES)]
        return carry

    lax.fori_loop(0, NCHUNK, up, 0)

    groups = [(r, l0, gl) for r in range(RPW) for (l0, gl) in GRPS]
    ng = len(groups)
    ngr = len(GRPS)

    flags = {}
    for r in range(RPW):
        fv = mv[5, pl.ds(r * LP, LANES)]
        anyf = jnp.max(fv) > 0
        cand = jnp.where((io < ngr) & (fv > 0), io * GMAX, jnp.int32(BIG))
        l0f = jnp.minimum(jnp.min(cand), jnp.int32((ngr - 1) * GMAX))
        l0f = pl.multiple_of(l0f, 8)
        for gi in range(ngr):
            flags[r * ngr + gi] = _extract_i32(fv, gi)
        hbk = pltpu.make_async_copy(
            table_hbm.at[tloc.at[pl.ds(r * LP + l0f, 8)]],
            bkbuf.at[r], gsems[r])

        @pl.when(anyf)
        def _bk():
            hbk.start()
            hbk.wait()

    def gstart(i):
        r, l0, gl = groups[i]
        h = pltpu.make_async_copy(
            table_hbm.at[tloc.at[pl.ds(r * LP + l0, gl)]],
            buf2.at[i % NBUF, pl.ds(0, gl)], gsems[i % NBUF])

        @pl.when(flags[i] == 0)
        def _g():
            h.start()
        return h

    def gwait(i):
        @pl.when(flags[i] == 0)
        def _w():
            gh[i].wait()

    def sstart(i):
        r, l0, gl = groups[i]
        hn = pltpu.make_async_copy(
            buf2.at[i % NBUF, pl.ds(0, gl)],
            out_hbm.at[wid * RPW + r, pl.ds(l0, gl)], ssems[i % NBUF])

        @pl.when(flags[i] == 0)
        def _sn():
            hn.start()

        @pl.when(flags[i] != 0)
        def _sb():
            for k in range(gl // 8):
                pltpu.make_async_copy(
                    bkbuf.at[r],
                    out_hbm.at[wid * RPW + r, pl.ds(l0 + 8 * k, 8)],
                    ssems[i % NBUF]).start()
        return hn

    gh = {0: gstart(0), 1: gstart(1)}
    sh = {}
    for i in range(ng):
        gwait(i)
        if i + 2 < ng:
            if i - 1 >= 0:
                sh[i - 1].wait()
            gh[i + 2] = gstart(i + 2)
        sh[i] = sstart(i)
    for i in range(max(0, ng - 3), ng):
        sh[i].wait()

    def write_row_to_out(b_s, l_s, src):
        g8 = (l_s // 8) * 8
        rr = l_s - g8
        pltpu.sync_copy(out_hbm.at[b_s, pl.ds(g8, 8)], obuf8)
        for r in range(8):
            @pl.when(rr == r)
            def _cp():
                def ck(k, c):
                    o = k * LANES
                    obuf8[r, pl.ds(o, LANES)] = src[pl.ds(o, LANES)]
                    return c
                lax.fori_loop(0, DCH, ck, 0)
        pltpu.sync_copy(obuf8, out_hbm.at[b_s, pl.ds(g8, 8)])

    def handle_lane(cls_s, st_s, en_s, w0_s, b_s, l_s):
        @pl.when(cls_s == 1)
        def _single():
            s8 = (st_s // 8) * 8
            sr = st_s - s8
            pltpu.sync_copy(ernie_hbm.at[b_s, pl.ds(s8, 8)], obuf8)
            for r in range(8):
                @pl.when(sr == r)
                def _cp():
                    def ck(k, c):
                        o = k * LANES
                        qrow[pl.ds(o, LANES)] = obuf8[r, pl.ds(o, LANES)]
                        return c
                    lax.fori_loop(0, DCH, ck, 0)
            write_row_to_out(b_s, l_s, qrow)

        @pl.when(cls_s == 2)
        def _attn():
            pltpu.async_copy(
                table_hbm.at[jnp.full((LANES,), w0_s, jnp.int32)],
                ebuf, sem).wait()

            def qk(k, c):
                o = k * LANES
                qrow[pl.ds(o, LANES)] = ebuf[0, pl.ds(o, LANES)]
                acc[pl.ds(o, LANES)] = zero16
                return c
            lax.fori_loop(0, DCH, qk, 0)

            c0 = st_s // LANES
            c1 = (en_s - 1) // LANES

            def chunk(c, carry):
                m_s, z_s = carry
                pltpu.sync_copy(ernie_hbm.at[b_s, pl.ds(c * LANES, LANES)],
                                ebuf)
                pos = c * LANES + io
                valid = (pos >= st_s) & (pos < en_s)
                sv = jnp.full((LANES,), -1e30, jnp.float32)
                for p in range(LANES):
                    def dk(k, pv):
                        o = k * LANES
                        return pv + (ebuf[p, pl.ds(o, LANES)]
                                     * qrow[pl.ds(o, LANES)])
                    part = lax.fori_loop(0, DCH, dk, zero16)
                    sp = jnp.sum(part)
                    sv = jnp.where(io == p, sp, sv)
                sv = jnp.where(valid, sv, jnp.float32(-1e30))
                mc = jnp.max(sv)
                m_new = jnp.maximum(m_s, mc)
                pe = jnp.exp(sv - m_new)
                pe = jnp.where(valid, pe, jnp.float32(0))
                ssum = jnp.sum(pe)
                scale_v = jnp.exp(jnp.full((LANES,), m_s - m_new))
                z_new = z_s * jnp.max(scale_v) + ssum

                def sk(k, c2):
                    o = k * LANES
                    acc[pl.ds(o, LANES)] = acc[pl.ds(o, LANES)] * scale_v
                    return c2
                lax.fori_loop(0, DCH, sk, 0)
                for p in range(LANES):
                    wp = _extract_f32(pe, p)

                    def ak(k, c3):
                        o = k * LANES
                        acc[pl.ds(o, LANES)] = (
                            acc[pl.ds(o, LANES)]
                            + ebuf[p, pl.ds(o, LANES)] * wp)
                        return c3
                    lax.fori_loop(0, DCH, ak, 0)
                return (m_new, z_new)

            _, z_fin = lax.fori_loop(
                c0, c1 + 1, chunk, (jnp.float32(-1e30), jnp.float32(0)))
            zinv_v = jnp.ones((LANES,), jnp.float32) / jnp.full(
                (LANES,), z_fin)

            def nk(k, c4):
                o = k * LANES
                acc[pl.ds(o, LANES)] = acc[pl.ds(o, LANES)] * zinv_v
                return c4
            lax.fori_loop(0, DCH, nk, 0)
            write_row_to_out(b_s, l_s, acc)

    def chunk_scan(ch, carry):
        cvec = clsl[pl.ds(ch * LANES, LANES)]

        @pl.when(jnp.max(cvec) > 0)
        def _special_chunk():
            svec = stl[pl.ds(ch * LANES, LANES)]
            evec = enl[pl.ds(ch * LANES, LANES)]
            wvec = w0l[pl.ds(ch * LANES, LANES)]

            def lane(p, c):
                cls_s = _extract_i32(cvec, p)

                @pl.when(cls_s > 0)
                def _go():
                    st_s = _extract_i32(svec, p)
                    en_s = _extract_i32(evec, p)
                    w0_s = _extract_i32(wvec, p)
                    lidx = ch * LANES + p
                    b_s = wid * RPW + lidx // LP
                    l_s = lidx % LP
                    handle_lane(cls_s, st_s, en_s, w0_s, b_s, l_s)
                return c
            lax.fori_loop(0, LANES, lane, 0)
        return carry

    lax.fori_loop(0, NCHUNK, chunk_scan, 0)


def _make_call():
    mesh = plsc.VectorSubcoreMesh(
        core_axis_name="c", subcore_axis_name="s",
        num_cores=NC, num_subcores=NS)

    @functools.partial(
        pl.kernel,
        out_type=jax.ShapeDtypeStruct((B, LP, D), jnp.float32),
        mesh=mesh,
        compiler_params=pltpu.CompilerParams(
            use_tc_tiling_on_sc=True, needs_layout_passes=False),
        scratch_types=[
            pltpu.VMEM((8, EPW_PAD), jnp.int32),
            pltpu.VMEM((EPW_PAD,), jnp.int32),
            pltpu.VMEM((EPW_PAD,), jnp.int32),
            pltpu.VMEM((EPW_PAD,), jnp.int32),
            pltpu.VMEM((EPW_PAD,), jnp.int32),
            pltpu.VMEM((EPW_PAD,), jnp.int32),
            pltpu.VMEM((NBUF, GMAX, D), jnp.float32),
            pltpu.VMEM((RPW, 8, D), jnp.float32),
            pltpu.VMEM((LANES, D), jnp.float32),
            pltpu.VMEM((8, D), jnp.float32),
            pltpu.VMEM((D,), jnp.float32),
            pltpu.VMEM((D,), jnp.float32),
            pltpu.SemaphoreType.DMA,
            pltpu.SemaphoreType.DMA,
            pltpu.SemaphoreType.DMA,
            pltpu.SemaphoreType.DMA,
            pltpu.SemaphoreType.DMA,
            pltpu.SemaphoreType.DMA,
            pltpu.SemaphoreType.DMA,
        ],
    )
    def call(ernie_hbm, meta_hbm, table_hbm, out_hbm, *scratch):
        _sc_body(ernie_hbm, meta_hbm, table_hbm, out_hbm, *scratch)

    return call


_sc_call = _make_call()


def kernel(ernie_output, word_index, table):
    w0 = word_index[:, :, 0]
    start = word_index[:, :, 1]
    end = word_index[:, :, 2]
    span = end - start

    is_br = (end < S) & (span <= 0)
    has_break = jnp.any(is_br, axis=1)
    jb = jnp.argmax(is_br, axis=1)
    jidx = jnp.arange(L, dtype=jnp.int32)[None, :]
    use_break = has_break[:, None] & (jidx >= jb[:, None])
    w0b = w0[jnp.arange(B), jb]

    notb = ~use_break
    attn = notb & (end < S) & (span > 1)
    single = notb & (end < S) & (span == 1)
    cls = attn.astype(jnp.int32) * 2 + single.astype(jnp.int32)

    tidx = jnp.where(use_break, w0b[:, None], w0)
    tidx = jnp.where(cls > 0, 0, tidx).astype(jnp.int32)
    startc = jnp.clip(start, 0, S - 1).astype(jnp.int32)

    def shape_w(a, padblk=None):
        a = a.astype(jnp.int32)
        if padblk is None:
            padblk = jnp.zeros((B, LP - L), jnp.int32)
        return jnp.concatenate([a, padblk], axis=1).reshape(
            NW, RPW, LP).reshape(NW, EPW_PAD)

    tpad = jnp.where(has_break[:, None], w0b[:, None],
                     jnp.arange(L, LP, dtype=jnp.int32)[None, :])
    gi = jnp.arange(len(GRPS), dtype=jnp.int32)
    flg = (has_break[:, None] & (jb[:, None] <= gi[None, :] * GMAX))
    flgrow = jnp.pad(flg.astype(jnp.int32), ((0, 0), (0, LP - len(GRPS))))
    flgrow = flgrow.reshape(NW, RPW, LP).reshape(NW, EPW_PAD)

    z = jnp.zeros((NW, EPW_PAD), jnp.int32)
    meta = jnp.stack(
        [shape_w(tidx, tpad), shape_w(cls), shape_w(startc), shape_w(end),
         shape_w(w0), flgrow, z, z], axis=1)

    return _sc_call(ernie_output, meta, table)[:, :L, :]
